# baseline (speedup 1.0000x reference)
import functools

import jax
import jax.numpy as jnp
from jax import lax
from jax.experimental import pallas as pl
from jax.experimental.pallas import tpu as pltpu
from jax.experimental.pallas import tpu_sc as plsc

N = 10000
H = 64
HH = 2 * H
NC = 2
NS = 16
L = 16
NW = NC * NS
CHUNK = 128
PAD_ROWS = 96
NPAD = N + PAD_ROWS
ROWS_PER_TILE = NPAD // NS


def _sc_aggregate(h, ein, ctab, nblk):
    mesh = plsc.VectorSubcoreMesh(core_axis_name="c", subcore_axis_name="s",
                                  num_cores=NC, num_subcores=NS)

    @functools.partial(
        pl.kernel,
        out_type=jax.ShapeDtypeStruct((NC, NPAD, HH), jnp.float32),
        mesh=mesh,
        compiler_params=pltpu.CompilerParams(needs_layout_passes=False,
                                             use_tc_tiling_on_sc=False),
        scratch_types=[
            pltpu.VMEM((2, 3, CHUNK), jnp.int32),
            pltpu.VMEM((2, 1, CHUNK), jnp.int32),
            pltpu.VMEM((2, CHUNK, H), jnp.float32),
            pltpu.VMEM((2, CHUNK, HH), jnp.float32),
            pltpu.VMEM((32,), jnp.float32),
            pltpu.VMEM((32,), jnp.float32),
            pltpu.VMEM_SHARED((NPAD, HH), jnp.float32),
            pltpu.SemaphoreType.DMA,
            pltpu.SemaphoreType.DMA,
            pltpu.SemaphoreType.DMA,
            pltpu.SemaphoreType.DMA,
            pltpu.SemaphoreType.DMA,
            pltpu.SemaphoreType.DMA,
        ],
    )
    def k(h_hbm, ein_hbm, ctab_hbm, out_hbm,
          ein_v, dst_v, rows_v, msg_v, c0tab_v, c1tab_v, acc_sh,
          isem0, isem1, gsem0, gsem1, ssem0, ssem1):
        isem = (isem0, isem1)
        gsem = (gsem0, gsem1)
        ssem = (ssem0, ssem1)
        cid = lax.axis_index("c")
        sid = lax.axis_index("s")
        wid = sid * NC + cid
        base = wid * nblk

        pltpu.sync_copy(ctab_hbm.at[0], c0tab_v)
        pltpu.sync_copy(ctab_hbm.at[1], c1tab_v)

        zv = jnp.zeros((L,), jnp.float32)

        def zrow_body(i, carry):
            for j in range(HH // L):
                msg_v[0, i, pl.ds(j * L, L)] = zv
            return carry
        lax.fori_loop(0, CHUNK, zrow_body, 0)
        row0 = sid * ROWS_PER_TILE
        for t in range(ROWS_PER_TILE // CHUNK):
            pltpu.sync_copy(msg_v.at[0],
                            acc_sh.at[pl.ds(row0 + t * CHUNK, CHUNK)])
        zrem = ROWS_PER_TILE % CHUNK
        if zrem:
            pltpu.sync_copy(
                msg_v.at[0, pl.ds(0, zrem)],
                acc_sh.at[pl.ds(row0 + ROWS_PER_TILE - zrem, zrem)])
        plsc.subcore_barrier()

        def idx_start(c, p):
            pltpu.async_copy(ein_hbm.at[base + c], ein_v.at[p], isem[p])

        def idx_wait(p):
            pltpu.make_async_copy(ein_hbm.at[base], ein_v.at[p], isem[p]).wait()

        def gather_start(p):
            pltpu.async_copy(h_hbm.at[ein_v.at[p, 0]], rows_v.at[p], gsem[p])

        def gather_wait(p):
            pltpu.make_async_copy(h_hbm.at[ein_v.at[p, 0]], rows_v.at[p],
                                  gsem[p]).wait()

        def scatter_start(p):
            pltpu.async_copy(msg_v.at[p], acc_sh.at[dst_v.at[p, 0]], ssem[p],
                             add=True)

        def scatter_wait(p):
            pltpu.make_async_copy(msg_v.at[p], acc_sh.at[dst_v.at[p, 0]],
                                  ssem[p]).wait()

        idx_start(0, 0)
        idx_start(1, 1)
        idx_wait(0)
        gather_start(0)

        def compute(p):
            @plsc.parallel_loop(0, CHUNK // L, unroll=4)
            def _(g):
                b16 = g * L
                ets = ein_v[p, 2, pl.ds(b16, L)]
                dst_v[p, 0, pl.ds(b16, L)] = ein_v[p, 1, pl.ds(b16, L)]
                c0g = plsc.load_gather(c0tab_v, [ets])
                c1g = plsc.load_gather(c1tab_v, [ets])
                for l in range(L):
                    c0 = c0g[l]
                    c1 = c1g[l]
                    for j in range(H // L):
                        v = rows_v[p, b16 + l, pl.ds(j * L, L)]
                        msg_v[p, b16 + l, pl.ds(j * L, L)] = v * c0
                        msg_v[p, b16 + l, pl.ds(H + j * L, L)] = v * c1

        def pair_body(c2, carry):
            for p in (0, 1):
                q = 1 - p
                c = 2 * c2 + p
                @pl.when(c + 1 < nblk)
                def _():
                    idx_wait(q)
                    gather_start(q)
                gather_wait(p)

                @pl.when(c >= 2)
                def _():
                    scatter_wait(p)
                compute(p)
                scatter_start(p)

                @pl.when(c + 2 < nblk)
                def _():
                    idx_start(c + 2, p)
            return carry
        lax.fori_loop(0, nblk // 2, pair_body, 0)
        scatter_wait(0)
        scatter_wait(1)

        plsc.subcore_barrier()
        pltpu.sync_copy(acc_sh.at[pl.ds(row0, ROWS_PER_TILE)],
                        out_hbm.at[cid, pl.ds(row0, ROWS_PER_TILE)])

    return k(h, ein, ctab)


def _tc_dense(h, accp, wself, sbases, bias):
    blk = 1000

    def body(h_ref, a_ref, w_ref, b_ref, bias_ref, o_ref):
        acc = a_ref[0] + a_ref[1]
        y = jnp.dot(h_ref[...], w_ref[...], preferred_element_type=jnp.float32)
        y = y + jnp.dot(acc, b_ref[...], preferred_element_type=jnp.float32)
        y = y + bias_ref[...]
        o_ref[...] = jnp.maximum(y, 0.0)

    return pl.pallas_call(
        body,
        grid=(N // blk,),
        in_specs=[
            pl.BlockSpec((blk, H), lambda i: (i, 0)),
            pl.BlockSpec((NC, blk, HH), lambda i: (0, i, 0)),
            pl.BlockSpec((H, H), lambda i: (0, 0)),
            pl.BlockSpec((HH, H), lambda i: (0, 0)),
            pl.BlockSpec((1, H), lambda i: (0, 0)),
        ],
        out_specs=pl.BlockSpec((blk, H), lambda i: (i, 0)),
        out_shape=jax.ShapeDtypeStruct((N, H), jnp.float32),
    )(h, accp, wself, sbases, bias.reshape(1, H))


def kernel(node_feat, edge_index, edge_type, embed,
           bases0, comp0, self0, bias0,
           bases1, comp1, self1, bias1):
    src = edge_index[0].astype(jnp.int32)
    dst = edge_index[1].astype(jnp.int32)
    et = edge_type.astype(jnp.int32)
    e = src.shape[0]
    quant = NW * CHUNK * 2
    e_pad = ((e + quant - 1) // quant) * quant
    pad = e_pad - e
    if pad:
        src = jnp.concatenate([src, jnp.zeros((pad,), jnp.int32)])
        dst = jnp.concatenate([dst, N + (jnp.arange(pad, dtype=jnp.int32) % PAD_ROWS)])
        et = jnp.concatenate([et, jnp.zeros((pad,), jnp.int32)])
    nblk = e_pad // (NW * CHUNK)
    ein = jnp.stack([src.reshape(-1, CHUNK), dst.reshape(-1, CHUNK),
                     et.reshape(-1, CHUNK)], axis=1)

    h = jnp.concatenate([embed, node_feat], axis=1)
    for bases, comp, wself, bias in ((bases0, comp0, self0, bias0),
                                     (bases1, comp1, self1, bias1)):
        r = comp.shape[0]
        ctab = jnp.zeros((2, 32), jnp.float32).at[:, :r].set(comp.T)
        accp = _sc_aggregate(h, ein, ctab, nblk)
        sb = jnp.concatenate([bases[0], bases[1]], axis=0)
        h = _tc_dense(h, accp, wself, sb, bias)
    return h


# speedup vs baseline: 1.0930x; 1.0930x over previous
import functools

import jax
import jax.numpy as jnp
from jax import lax
from jax.experimental import pallas as pl
from jax.experimental.pallas import tpu as pltpu
from jax.experimental.pallas import tpu_sc as plsc

N = 10000
H = 64
HH = 2 * H
NC = 2
NS = 16
L = 16
NW = NC * NS
CHUNK = 128
PAD_ROWS = 96
NPAD = N + PAD_ROWS
ROWS_PER_TILE = NPAD // NS


def _sc_aggregate(h, ein, ctab, nblk):
    mesh = plsc.VectorSubcoreMesh(core_axis_name="c", subcore_axis_name="s",
                                  num_cores=NC, num_subcores=NS)

    @functools.partial(
        pl.kernel,
        out_type=jax.ShapeDtypeStruct((NC, NPAD, HH), jnp.float32),
        mesh=mesh,
        compiler_params=pltpu.CompilerParams(needs_layout_passes=False,
                                             use_tc_tiling_on_sc=False),
        scratch_types=[
            pltpu.VMEM((2, 3, CHUNK), jnp.int32),
            pltpu.VMEM((2, 1, CHUNK), jnp.int32),
            pltpu.VMEM((2, CHUNK, H), jnp.float32),
            pltpu.VMEM((2, CHUNK, HH), jnp.float32),
            pltpu.VMEM((32,), jnp.float32),
            pltpu.VMEM((32,), jnp.float32),
            pltpu.VMEM_SHARED((NPAD, HH), jnp.float32),
            pltpu.SemaphoreType.DMA,
            pltpu.SemaphoreType.DMA,
            pltpu.SemaphoreType.DMA,
            pltpu.SemaphoreType.DMA,
            pltpu.SemaphoreType.DMA,
            pltpu.SemaphoreType.DMA,
        ],
    )
    def k(h_hbm, ein_hbm, ctab_hbm, out_hbm,
          ein_v, dst_v, rows_v, msg_v, c0tab_v, c1tab_v, acc_sh,
          isem0, isem1, gsem0, gsem1, ssem0, ssem1):
        isem = (isem0, isem1)
        gsem = (gsem0, gsem1)
        ssem = (ssem0, ssem1)
        cid = lax.axis_index("c")
        sid = lax.axis_index("s")
        wid = sid * NC + cid
        base = wid * nblk

        pltpu.sync_copy(ctab_hbm.at[0], c0tab_v)
        pltpu.sync_copy(ctab_hbm.at[1], c1tab_v)

        zv = jnp.zeros((L,), jnp.float32)

        def zrow_body(i, carry):
            for j in range(HH // L):
                msg_v[0, i, pl.ds(j * L, L)] = zv
            return carry
        lax.fori_loop(0, CHUNK, zrow_body, 0)
        row0 = sid * ROWS_PER_TILE
        for t in range(ROWS_PER_TILE // CHUNK):
            pltpu.sync_copy(msg_v.at[0],
                            acc_sh.at[pl.ds(row0 + t * CHUNK, CHUNK)])
        zrem = ROWS_PER_TILE % CHUNK
        if zrem:
            pltpu.sync_copy(
                msg_v.at[0, pl.ds(0, zrem)],
                acc_sh.at[pl.ds(row0 + ROWS_PER_TILE - zrem, zrem)])
        plsc.subcore_barrier()

        def idx_start(c, p):
            pltpu.async_copy(ein_hbm.at[base + c], ein_v.at[p], isem[p])

        def idx_wait(p):
            pltpu.make_async_copy(ein_hbm.at[base], ein_v.at[p], isem[p]).wait()

        HC = CHUNK // 2

        def gather_start(p):
            pltpu.async_copy(h_hbm.at[ein_v.at[p, 0, pl.ds(0, HC)]],
                             rows_v.at[p, pl.ds(0, HC)], gsem[p])
            pltpu.async_copy(h_hbm.at[ein_v.at[p, 0, pl.ds(HC, HC)]],
                             rows_v.at[p, pl.ds(HC, HC)], gsem[p])

        def gather_wait(p):
            pltpu.make_async_copy(h_hbm.at[ein_v.at[p, 0, pl.ds(0, HC)]],
                                  rows_v.at[p, pl.ds(0, HC)], gsem[p]).wait()
            pltpu.make_async_copy(h_hbm.at[ein_v.at[p, 0, pl.ds(HC, HC)]],
                                  rows_v.at[p, pl.ds(HC, HC)], gsem[p]).wait()

        def scatter_start(p):
            pltpu.async_copy(msg_v.at[p], acc_sh.at[dst_v.at[p, 0]], ssem[p],
                             add=True)

        def scatter_wait(p):
            pltpu.make_async_copy(msg_v.at[p], acc_sh.at[dst_v.at[p, 0]],
                                  ssem[p]).wait()

        idx_start(0, 0)
        idx_start(1, 1)
        idx_wait(0)
        gather_start(0)

        def compute(p):
            @plsc.parallel_loop(0, CHUNK // L, unroll=2)
            def _(g):
                b16 = g * L
                ets = ein_v[p, 2, pl.ds(b16, L)]
                dst_v[p, 0, pl.ds(b16, L)] = ein_v[p, 1, pl.ds(b16, L)]
                c0g = plsc.load_gather(c0tab_v, [ets])
                c1g = plsc.load_gather(c1tab_v, [ets])
                for l in range(L):
                    c0 = c0g[l]
                    c1 = c1g[l]
                    for j in range(H // L):
                        v = rows_v[p, b16 + l, pl.ds(j * L, L)]
                        msg_v[p, b16 + l, pl.ds(j * L, L)] = v * c0
                        msg_v[p, b16 + l, pl.ds(H + j * L, L)] = v * c1

        def pair_body(c2, carry):
            for p in (0, 1):
                q = 1 - p
                c = 2 * c2 + p
                @pl.when(c + 1 < nblk)
                def _():
                    idx_wait(q)
                    gather_start(q)
                gather_wait(p)

                @pl.when(c >= 2)
                def _():
                    scatter_wait(p)
                compute(p)
                scatter_start(p)

                @pl.when(c + 2 < nblk)
                def _():
                    idx_start(c + 2, p)
            return carry
        lax.fori_loop(0, nblk // 2, pair_body, 0)
        scatter_wait(0)
        scatter_wait(1)

        plsc.subcore_barrier()
        pltpu.sync_copy(acc_sh.at[pl.ds(row0, ROWS_PER_TILE)],
                        out_hbm.at[cid, pl.ds(row0, ROWS_PER_TILE)])

    return k(h, ein, ctab)


def _tc_dense(h, accp, wself, sbases, bias):
    blk = 1000

    def body(h_ref, a_ref, w_ref, b_ref, bias_ref, o_ref):
        acc = a_ref[0] + a_ref[1]
        y = jnp.dot(h_ref[...], w_ref[...], preferred_element_type=jnp.float32)
        y = y + jnp.dot(acc, b_ref[...], preferred_element_type=jnp.float32)
        y = y + bias_ref[...]
        o_ref[...] = jnp.maximum(y, 0.0)

    return pl.pallas_call(
        body,
        grid=(N // blk,),
        in_specs=[
            pl.BlockSpec((blk, H), lambda i: (i, 0)),
            pl.BlockSpec((NC, blk, HH), lambda i: (0, i, 0)),
            pl.BlockSpec((H, H), lambda i: (0, 0)),
            pl.BlockSpec((HH, H), lambda i: (0, 0)),
            pl.BlockSpec((1, H), lambda i: (0, 0)),
        ],
        out_specs=pl.BlockSpec((blk, H), lambda i: (i, 0)),
        out_shape=jax.ShapeDtypeStruct((N, H), jnp.float32),
    )(h, accp, wself, sbases, bias.reshape(1, H))


def kernel(node_feat, edge_index, edge_type, embed,
           bases0, comp0, self0, bias0,
           bases1, comp1, self1, bias1):
    src = edge_index[0].astype(jnp.int32)
    dst = edge_index[1].astype(jnp.int32)
    et = edge_type.astype(jnp.int32)
    e = src.shape[0]
    quant = NW * CHUNK * 2
    e_pad = ((e + quant - 1) // quant) * quant
    pad = e_pad - e
    if pad:
        src = jnp.concatenate([src, jnp.zeros((pad,), jnp.int32)])
        dst = jnp.concatenate([dst, N + (jnp.arange(pad, dtype=jnp.int32) % PAD_ROWS)])
        et = jnp.concatenate([et, jnp.zeros((pad,), jnp.int32)])
    nblk = e_pad // (NW * CHUNK)
    ein = jnp.stack([src.reshape(-1, CHUNK), dst.reshape(-1, CHUNK),
                     et.reshape(-1, CHUNK)], axis=1)

    h = jnp.concatenate([embed, node_feat], axis=1)
    for bases, comp, wself, bias in ((bases0, comp0, self0, bias0),
                                     (bases1, comp1, self1, bias1)):
        r = comp.shape[0]
        ctab = jnp.zeros((2, 32), jnp.float32).at[:, :r].set(comp.T)
        accp = _sc_aggregate(h, ein, ctab, nblk)
        sb = jnp.concatenate([bases[0], bases[1]], axis=0)
        h = _tc_dense(h, accp, wself, sb, bias)
    return h


# speedup vs baseline: 1.0932x; 1.0002x over previous
import functools

import jax
import jax.numpy as jnp
from jax import lax
from jax.experimental import pallas as pl
from jax.experimental.pallas import tpu as pltpu
from jax.experimental.pallas import tpu_sc as plsc

N = 10000
H = 64
HH = 2 * H
NC = 2
NS = 16
L = 16
NW = NC * NS
CHUNK = 128
PAD_ROWS = 96
NPAD = N + PAD_ROWS
ROWS_PER_TILE = NPAD // NS


def _sc_aggregate(h, ein, ctab, nblk):
    mesh = plsc.VectorSubcoreMesh(core_axis_name="c", subcore_axis_name="s",
                                  num_cores=NC, num_subcores=NS)

    @functools.partial(
        pl.kernel,
        out_type=jax.ShapeDtypeStruct((NC, NPAD, HH), jnp.float32),
        mesh=mesh,
        compiler_params=pltpu.CompilerParams(needs_layout_passes=False,
                                             use_tc_tiling_on_sc=False),
        scratch_types=[
            pltpu.VMEM((2, 3, CHUNK), jnp.int32),
            pltpu.VMEM((2, 1, CHUNK), jnp.int32),
            pltpu.VMEM((2, CHUNK, H), jnp.float32),
            pltpu.VMEM((2, CHUNK, HH), jnp.float32),
            pltpu.VMEM((32,), jnp.float32),
            pltpu.VMEM((32,), jnp.float32),
            pltpu.VMEM_SHARED((NPAD, HH), jnp.float32),
            pltpu.SemaphoreType.DMA,
            pltpu.SemaphoreType.DMA,
            pltpu.SemaphoreType.DMA,
            pltpu.SemaphoreType.DMA,
            pltpu.SemaphoreType.DMA,
            pltpu.SemaphoreType.DMA,
        ],
    )
    def k(h_hbm, ein_hbm, ctab_hbm, out_hbm,
          ein_v, dst_v, rows_v, msg_v, c0tab_v, c1tab_v, acc_sh,
          isem0, isem1, gsem0, gsem1, ssem0, ssem1):
        isem = (isem0, isem1)
        gsem = (gsem0, gsem1)
        ssem = (ssem0, ssem1)
        cid = lax.axis_index("c")
        sid = lax.axis_index("s")
        wid = sid * NC + cid
        base = wid * nblk

        pltpu.sync_copy(ctab_hbm.at[0], c0tab_v)
        pltpu.sync_copy(ctab_hbm.at[1], c1tab_v)

        zv = jnp.zeros((L,), jnp.float32)

        def zrow_body(i, carry):
            for j in range(HH // L):
                msg_v[0, i, pl.ds(j * L, L)] = zv
            return carry
        lax.fori_loop(0, CHUNK, zrow_body, 0)
        row0 = sid * ROWS_PER_TILE
        for t in range(ROWS_PER_TILE // CHUNK):
            pltpu.sync_copy(msg_v.at[0],
                            acc_sh.at[pl.ds(row0 + t * CHUNK, CHUNK)])
        zrem = ROWS_PER_TILE % CHUNK
        if zrem:
            pltpu.sync_copy(
                msg_v.at[0, pl.ds(0, zrem)],
                acc_sh.at[pl.ds(row0 + ROWS_PER_TILE - zrem, zrem)])
        plsc.subcore_barrier()

        def idx_start(c, p):
            pltpu.async_copy(ein_hbm.at[base + c], ein_v.at[p], isem[p])

        def idx_wait(p):
            pltpu.make_async_copy(ein_hbm.at[base], ein_v.at[p], isem[p]).wait()

        def gather_start(p):
            pltpu.async_copy(h_hbm.at[ein_v.at[p, 0]], rows_v.at[p], gsem[p])

        def gather_wait(p):
            pltpu.make_async_copy(h_hbm.at[ein_v.at[p, 0]], rows_v.at[p],
                                  gsem[p]).wait()

        def scatter_start(p):
            pltpu.async_copy(msg_v.at[p], acc_sh.at[dst_v.at[p, 0]], ssem[p],
                             add=True)

        def scatter_wait(p):
            pltpu.make_async_copy(msg_v.at[p], acc_sh.at[dst_v.at[p, 0]],
                                  ssem[p]).wait()

        idx_start(0, 0)
        idx_start(1, 1)
        idx_wait(0)
        gather_start(0)

        def compute(p):
            @plsc.parallel_loop(0, CHUNK // L, unroll=2)
            def _(g):
                b16 = g * L
                ets = ein_v[p, 2, pl.ds(b16, L)]
                dst_v[p, 0, pl.ds(b16, L)] = ein_v[p, 1, pl.ds(b16, L)]
                c0g = plsc.load_gather(c0tab_v, [ets])
                c1g = plsc.load_gather(c1tab_v, [ets])
                for l in range(L):
                    c0 = c0g[l]
                    c1 = c1g[l]
                    for j in range(H // L):
                        v = rows_v[p, b16 + l, pl.ds(j * L, L)]
                        msg_v[p, b16 + l, pl.ds(j * L, L)] = v * c0
                        msg_v[p, b16 + l, pl.ds(H + j * L, L)] = v * c1

        def pair_body(c2, carry):
            for p in (0, 1):
                q = 1 - p
                c = 2 * c2 + p
                @pl.when(c + 1 < nblk)
                def _():
                    idx_wait(q)
                    gather_start(q)
                gather_wait(p)

                @pl.when(c >= 2)
                def _():
                    scatter_wait(p)
                compute(p)
                scatter_start(p)

                @pl.when(c + 2 < nblk)
                def _():
                    idx_start(c + 2, p)
            return carry
        lax.fori_loop(0, nblk // 2, pair_body, 0)
        scatter_wait(0)
        scatter_wait(1)

        plsc.subcore_barrier()
        pltpu.sync_copy(acc_sh.at[pl.ds(row0, ROWS_PER_TILE)],
                        out_hbm.at[cid, pl.ds(row0, ROWS_PER_TILE)])

    return k(h, ein, ctab)


def _tc_dense(h, accp, wself, sbases, bias):
    blk = 1000

    def body(h_ref, a_ref, w_ref, b_ref, bias_ref, o_ref):
        acc = a_ref[0] + a_ref[1]
        y = jnp.dot(h_ref[...], w_ref[...], preferred_element_type=jnp.float32)
        y = y + jnp.dot(acc, b_ref[...], preferred_element_type=jnp.float32)
        y = y + bias_ref[...]
        o_ref[...] = jnp.maximum(y, 0.0)

    return pl.pallas_call(
        body,
        grid=(N // blk,),
        in_specs=[
            pl.BlockSpec((blk, H), lambda i: (i, 0)),
            pl.BlockSpec((NC, blk, HH), lambda i: (0, i, 0)),
            pl.BlockSpec((H, H), lambda i: (0, 0)),
            pl.BlockSpec((HH, H), lambda i: (0, 0)),
            pl.BlockSpec((1, H), lambda i: (0, 0)),
        ],
        out_specs=pl.BlockSpec((blk, H), lambda i: (i, 0)),
        out_shape=jax.ShapeDtypeStruct((N, H), jnp.float32),
    )(h, accp, wself, sbases, bias.reshape(1, H))


def kernel(node_feat, edge_index, edge_type, embed,
           bases0, comp0, self0, bias0,
           bases1, comp1, self1, bias1):
    src = edge_index[0].astype(jnp.int32)
    dst = edge_index[1].astype(jnp.int32)
    et = edge_type.astype(jnp.int32)
    e = src.shape[0]
    quant = NW * CHUNK * 2
    e_pad = ((e + quant - 1) // quant) * quant
    pad = e_pad - e
    if pad:
        src = jnp.concatenate([src, jnp.zeros((pad,), jnp.int32)])
        dst = jnp.concatenate([dst, N + (jnp.arange(pad, dtype=jnp.int32) % PAD_ROWS)])
        et = jnp.concatenate([et, jnp.zeros((pad,), jnp.int32)])
    nblk = e_pad // (NW * CHUNK)
    ein = jnp.stack([src.reshape(-1, CHUNK), dst.reshape(-1, CHUNK),
                     et.reshape(-1, CHUNK)], axis=1)

    h = jnp.concatenate([embed, node_feat], axis=1)
    for bases, comp, wself, bias in ((bases0, comp0, self0, bias0),
                                     (bases1, comp1, self1, bias1)):
        r = comp.shape[0]
        ctab = jnp.zeros((2, 32), jnp.float32).at[:, :r].set(comp.T)
        accp = _sc_aggregate(h, ein, ctab, nblk)
        sb = jnp.concatenate([bases[0], bases[1]], axis=0)
        h = _tc_dense(h, accp, wself, sb, bias)
    return h
